# baseline (speedup 1.0000x reference)
import functools

import jax
import jax.numpy as jnp
from jax.experimental import pallas as pl
from jax.experimental.pallas import tpu as pltpu


_NEG_HUGE = -1.0e30


def _round_up(x, m):
    return (x + m - 1) // m * m


def _loss_kernel(emb_ref, w_ref, b_ref, tgt_ref, m_out, l_out, t_out,
                 s_sc, *, tv, cw):
    i = pl.program_id(0)
    j = pl.program_id(1)

    @pl.when(j == 0)
    def _():
        s_sc[:, 0:1] = jnp.full_like(s_sc[:, 0:1], -jnp.inf)
        s_sc[:, 1:3] = jnp.zeros_like(s_sc[:, 1:3])

    m = s_sc[:, 0:1]
    l = s_sc[:, 1:2]
    t = s_sc[:, 2:3]
    tgt = tgt_ref[0, 0, :]
    bn = s_sc.shape[0]
    base = j * tv
    col = jax.lax.broadcasted_iota(jnp.int32, (bn, cw), 1)

    t_hist = []
    for k in range(tv // cw):
        wk = w_ref[:, k * cw:(k + 1) * cw].astype(jnp.bfloat16)
        if k >= 1:
            gate = (t_hist[k - 1][0, 0] * 0.0 + 1.0).astype(jnp.bfloat16)
            wk = wk * gate
        raw = jnp.dot(emb_ref[...], wk, preferred_element_type=jnp.float32)
        bk = b_ref[0:1, k * cw:(k + 1) * cw]

        match = col == (tgt[:, None] - (base + k * cw))
        t = t + jnp.sum(jnp.where(match, raw, 0.0), axis=-1, keepdims=True)
        t_hist.append(t)

        m_new = jnp.maximum(m, (raw + bk).max(axis=-1, keepdims=True))
        l = (l * jnp.exp(m - m_new)
             + jnp.sum(jnp.exp(raw + bk - m_new), axis=-1, keepdims=True))
        m = m_new

    s_sc[:, 0:1] = m
    s_sc[:, 1:2] = l
    s_sc[:, 2:3] = t

    @pl.when(j == pl.num_programs(1) - 1)
    def _():
        m_out[0, 0, :] = m[:, 0]
        l_out[0, 0, :] = l[:, 0]
        t_out[0, 0, :] = t[:, 0]


def kernel(embeddings, softmax_w, softmax_b, targets):
    n, d = embeddings.shape
    d2, v = softmax_w.shape
    assert d == d2

    tv, cw = 1280, 640
    if v % tv != 0:
        for cand_tv, cand_cw in ((1024, 512), (512, 256), (256, 128)):
            if v % cand_tv == 0:
                tv, cw = cand_tv, cand_cw
                break
    v_pad = _round_up(v, tv)

    n_pad = _round_up(max(n, 8), 256)
    bn = n_pad // 2 if n_pad % 512 == 0 else n_pad
    nb = n_pad // bn

    emb_p = embeddings
    if n_pad != n:
        emb_p = jnp.zeros((n_pad, d), embeddings.dtype).at[:n].set(embeddings)
    emb16 = emb_p.astype(jnp.bfloat16)

    if v_pad != v:
        w_p = jnp.zeros((d, v_pad), softmax_w.dtype).at[:, :v].set(softmax_w)
        b_p = jnp.full((v_pad,), _NEG_HUGE, jnp.float32).at[:v].set(
            softmax_b.astype(jnp.float32))
    else:
        w_p = softmax_w
        b_p = softmax_b.astype(jnp.float32)
    b2d = b_p.reshape(1, v_pad)

    tgt = jnp.zeros((n_pad,), jnp.int32).at[:n].set(targets.astype(jnp.int32))
    tgt3 = tgt.reshape(nb, 1, bn)

    out_shape = jax.ShapeDtypeStruct((nb, 1, bn), jnp.float32)
    m_h, l_h, t_h = pl.pallas_call(
        functools.partial(_loss_kernel, tv=tv, cw=cw),
        out_shape=(out_shape, out_shape, out_shape),
        grid_spec=pltpu.PrefetchScalarGridSpec(
            num_scalar_prefetch=0,
            grid=(nb, v_pad // tv),
            in_specs=[
                pl.BlockSpec((bn, d), lambda i, j: (i, 0)),
                pl.BlockSpec((d, tv), lambda i, j: (0, j)),
                pl.BlockSpec((1, tv), lambda i, j: (0, j)),
                pl.BlockSpec((1, 1, bn), lambda i, j: (i, 0, 0)),
            ],
            out_specs=(
                pl.BlockSpec((1, 1, bn), lambda i, j: (i, 0, 0)),
                pl.BlockSpec((1, 1, bn), lambda i, j: (i, 0, 0)),
                pl.BlockSpec((1, 1, bn), lambda i, j: (i, 0, 0)),
            ),
            scratch_shapes=[
                pltpu.VMEM((bn, 128), jnp.float32),
            ],
        ),
        compiler_params=pltpu.CompilerParams(
            dimension_semantics=("parallel", "arbitrary"),
            vmem_limit_bytes=64 * 1024 * 1024),
    )(emb16, w_p, b2d, tgt3)

    lse = m_h.reshape(n_pad)[:n] + jnp.log(l_h.reshape(n_pad)[:n])
    tgt_logits = (t_h.reshape(n_pad)[:n]
                  + softmax_b[targets.astype(jnp.int32)].astype(jnp.float32))
    return jnp.sum(lse - tgt_logits)


# speedup vs baseline: 1.9136x; 1.9136x over previous
import functools

import jax
import jax.numpy as jnp
from jax.experimental import pallas as pl
from jax.experimental.pallas import tpu as pltpu


_NEG_HUGE = -1.0e30


def _round_up(x, m):
    return (x + m - 1) // m * m


def _loss_kernel(emb_ref, w_ref, b_ref, tgt_ref, out_ref,
                 emb_sc, m_sc, l_sc, t_sc, *, tv):
    j = pl.program_id(1)

    @pl.when(j == 0)
    def _():
        emb_sc[...] = emb_ref[...].astype(jnp.bfloat16)
        m_sc[...] = jnp.full_like(m_sc, -jnp.inf)
        l_sc[...] = jnp.zeros_like(l_sc)
        t_sc[...] = jnp.zeros_like(t_sc)

    logits = jnp.dot(emb_sc[...], w_ref[...].astype(jnp.bfloat16),
                     preferred_element_type=jnp.float32) + b_ref[...]

    m_prev = m_sc[...]
    m_new = jnp.maximum(m_prev, logits.max(axis=-1, keepdims=True))
    l_sc[...] = (l_sc[...] * jnp.exp(m_prev - m_new)
                 + jnp.sum(jnp.exp(logits - m_new), axis=-1, keepdims=True))
    m_sc[...] = m_new

    bn = logits.shape[0]
    col = jax.lax.broadcasted_iota(jnp.int32, (bn, tv), 1)
    match = col == (tgt_ref[0, 0, :][:, None] - j * tv)
    t_sc[...] += jnp.sum(jnp.where(match, logits, 0.0), axis=-1, keepdims=True)

    @pl.when(j == pl.num_programs(1) - 1)
    def _():
        out_ref[...] = m_sc[...] + jnp.log(l_sc[...]) - t_sc[...]


def kernel(embeddings, softmax_w, softmax_b, targets):
    n, d = embeddings.shape
    d2, v = softmax_w.shape
    assert d == d2

    tv = 1280
    if v % tv != 0:
        for cand in (1024, 768, 512, 384, 256, 128):
            if v % cand == 0:
                tv = cand
                break
    v_pad = _round_up(v, tv)

    n_pad = _round_up(max(n, 8), 256)
    bn = n_pad // 2 if n_pad % 512 == 0 else n_pad
    nb = n_pad // bn

    emb_p = embeddings
    if n_pad != n:
        emb_p = jnp.zeros((n_pad, d), embeddings.dtype).at[:n].set(embeddings)
    if v_pad != v:
        w_p = jnp.zeros((d, v_pad), softmax_w.dtype).at[:, :v].set(softmax_w)
        b_p = jnp.full((v_pad,), _NEG_HUGE, jnp.float32).at[:v].set(
            softmax_b.astype(jnp.float32))
    else:
        w_p = softmax_w
        b_p = softmax_b.astype(jnp.float32)
    b2d = b_p.reshape(1, v_pad)

    tgt = jnp.zeros((n_pad,), jnp.int32).at[:n].set(targets.astype(jnp.int32))
    tgt3 = tgt.reshape(nb, 1, bn)

    per_row = pl.pallas_call(
        functools.partial(_loss_kernel, tv=tv),
        out_shape=jax.ShapeDtypeStruct((n_pad, 1), jnp.float32),
        grid_spec=pltpu.PrefetchScalarGridSpec(
            num_scalar_prefetch=0,
            grid=(nb, v_pad // tv),
            in_specs=[
                pl.BlockSpec((bn, d), lambda i, j: (i, 0)),
                pl.BlockSpec((d, tv), lambda i, j: (0, j)),
                pl.BlockSpec((1, tv), lambda i, j: (0, j)),
                pl.BlockSpec((1, 1, bn), lambda i, j: (i, 0, 0)),
            ],
            out_specs=pl.BlockSpec((bn, 1), lambda i, j: (i, 0)),
            scratch_shapes=[
                pltpu.VMEM((bn, d), jnp.bfloat16),
                pltpu.VMEM((bn, 1), jnp.float32),
                pltpu.VMEM((bn, 1), jnp.float32),
                pltpu.VMEM((bn, 1), jnp.float32),
            ],
        ),
        compiler_params=pltpu.CompilerParams(
            dimension_semantics=("parallel", "arbitrary"),
            vmem_limit_bytes=64 * 1024 * 1024),
    )(emb_p, w_p, b2d, tgt3)

    return jnp.sum(per_row[:n, 0])
